# baseline (speedup 1.0000x reference)
import functools

import jax
import jax.numpy as jnp
from jax import lax
from jax.experimental import pallas as pl
from jax.experimental.pallas import tpu as pltpu
from jax.experimental.pallas import tpu_sc as plsc

NC = 2
NS = 16
NW = NC * NS
L = 16
D = 32
IDX_CHUNK = 128


def _gmf_body(uidx_hbm, iidx_hbm, utab_hbm, itab_hbm, wb_hbm, out_hbm,
              uidx_v, iidx_v, urows_v, irows_v, wb_v, wbc_v, out_v,
              sem_u, sem_i):
    bpw = out_v.shape[0]
    nchunk = bpw // IDX_CHUNK
    wid = lax.axis_index("s") * NC + lax.axis_index("c")

    pltpu.sync_copy(uidx_hbm.at[pl.ds(wid * nchunk, nchunk)], uidx_v)
    pltpu.sync_copy(iidx_hbm.at[pl.ds(wid * nchunk, nchunk)], iidx_v)

    copies = []
    for k in range(nchunk):
        copies.append(pltpu.async_copy(
            utab_hbm.at[uidx_v.at[k]],
            urows_v.at[pl.ds(k * IDX_CHUNK, IDX_CHUNK)], sem_u))
        copies.append(pltpu.async_copy(
            itab_hbm.at[iidx_v.at[k]],
            irows_v.at[pl.ds(k * IDX_CHUNK, IDX_CHUNK)], sem_i))

    pltpu.sync_copy(wb_hbm, wb_v)
    for j in range(D):
        wbc_v[pl.ds(j * L, L)] = plsc.load_gather(
            wb_v, [jnp.full((L,), j, jnp.int32)])
    bvec = plsc.load_gather(wb_v, [jnp.full((L,), D, jnp.int32)])

    for c in copies:
        c.wait()

    lanes = lax.iota(jnp.int32, L)

    def group(g, carry):
        rows = lanes + g * L
        acc = bvec
        for j in range(D):
            cj = jnp.full((L,), j, jnp.int32)
            ucol = plsc.load_gather(urows_v, [rows, cj])
            icol = plsc.load_gather(irows_v, [rows, cj])
            wv = wbc_v[pl.ds(j * L, L)]
            acc = acc + ucol * icol * wv
        out_v[pl.ds(g * L, L)] = 1.0 / (1.0 + jnp.exp(-acc))
        return carry

    lax.fori_loop(0, bpw // L, group, 0)

    pltpu.sync_copy(out_v, out_hbm.at[pl.ds(wid * bpw, bpw)])


def kernel(user_indices, item_indices, user_table, item_table, W, b):
    B = user_indices.shape[0]
    bpw = B // NW
    nchunk = bpw // IDX_CHUNK

    wb = jnp.zeros((64,), jnp.float32)
    wb = wb.at[:D].set(W.reshape(-1).astype(jnp.float32))
    wb = wb.at[D].set(b.reshape(()).astype(jnp.float32))

    uidx = user_indices.astype(jnp.int32).reshape(NW * nchunk, IDX_CHUNK)
    iidx = item_indices.astype(jnp.int32).reshape(NW * nchunk, IDX_CHUNK)

    run = pl.kernel(
        _gmf_body,
        out_type=jax.ShapeDtypeStruct((B,), jnp.float32),
        mesh=plsc.VectorSubcoreMesh(
            core_axis_name="c", subcore_axis_name="s",
            num_cores=NC, num_subcores=NS),
        scratch_types=[
            pltpu.VMEM((nchunk, IDX_CHUNK), jnp.int32),
            pltpu.VMEM((nchunk, IDX_CHUNK), jnp.int32),
            pltpu.VMEM((bpw, D), jnp.float32),
            pltpu.VMEM((bpw, D), jnp.float32),
            pltpu.VMEM((64,), jnp.float32),
            pltpu.VMEM((D * L,), jnp.float32),
            pltpu.VMEM((bpw,), jnp.float32),
            pltpu.SemaphoreType.DMA,
            pltpu.SemaphoreType.DMA,
        ],
        compiler_params=pltpu.CompilerParams(
            needs_layout_passes=False, use_tc_tiling_on_sc=False),
    )
    out = run(uidx, iidx, user_table, item_table, wb)
    return out.reshape(B, 1)
